# Initial kernel scaffold; baseline (speedup 1.0000x reference)
import jax
import jax.numpy as jnp
from jax.experimental import pallas as pl


def kernel(patch_tokens, voxel_features, voxel_coords, image_sizes, K, Rt, W1, b1, W2, b2):
    raise NotImplementedError("write your pallas kernel here")


# speedup vs baseline: 4.2525x; 4.2525x over previous
import functools

import jax
import jax.numpy as jnp
from jax import lax
from jax.experimental import pallas as pl
from jax.experimental.pallas import tpu as pltpu
from jax.experimental.pallas import tpu_sc as plsc

_RESIZE = 518.0
_PATCH = 14.0
_GRID = 37


def _prep_body(pt_ref, pts_ref, k_ref, rt_ref, isz_ref, w1b_ref, b1_ref,
               table_ref, idx_ref):
    b = pl.program_id(0)
    tok = pt_ref[0]
    mean_tok = jnp.mean(tok, axis=0)
    table_ref[0] = (
        jnp.dot(mean_tok, w1b_ref[...], preferred_element_type=jnp.float32)
        + b1_ref[...]
    )

    ph = pts_ref[0].astype(jnp.bfloat16)
    rb = rt_ref[0].astype(jnp.bfloat16)
    cam = jnp.dot(rb, ph, preferred_element_type=jnp.float32)
    camb = cam[:3].astype(jnp.bfloat16)
    kb = k_ref[0].astype(jnp.bfloat16)
    pix = jnp.dot(kb, camb, preferred_element_type=jnp.float32)
    denom = pix[2:3, :] + 1e-12
    w_orig = isz_ref[0, 0].astype(jnp.float32)
    h_orig = isz_ref[0, 1].astype(jnp.float32)
    u = (pix[0:1, :] / denom) * (_RESIZE / w_orig)
    v = (pix[1:2, :] / denom) * (_RESIZE / h_orig)
    px = jnp.clip((u / _PATCH).astype(jnp.int32), 0, _GRID - 1)
    py = jnp.clip((v / _PATCH).astype(jnp.int32), 0, _GRID - 1)
    m = pt_ref.shape[2]
    idx_ref[0] = px * _GRID + py + b * m


def _mlp_body(g_ref, vox_ref, w1a_ref, w2_ref, b2_ref, out_ref):
    h = (jnp.dot(vox_ref[...], w1a_ref[...], preferred_element_type=jnp.float32)
         + g_ref[...])
    h = jnp.maximum(h, 0.0)
    out_ref[...] = (
        jnp.dot(h, w2_ref[...], preferred_element_type=jnp.float32)
        + b2_ref[...]
    )


_CHUNK = 128


def _sc_gather_body(table_hbm, idx_hbm, out_hbm, idx_v, rows_v, sems, n_chunks,
                    n_buf):
    wid = lax.axis_index("s") * 2 + lax.axis_index("c")
    base = wid * (n_chunks * _CHUNK)
    for c in range(n_chunks):
        s = c % n_buf
        pltpu.sync_copy(idx_hbm.at[pl.ds(base + c * _CHUNK, _CHUNK)],
                        idx_v.at[s])
        pltpu.async_copy(table_hbm.at[idx_v.at[s]], rows_v.at[s], sems.at[s])
        if c >= n_buf - 1:
            d = c - (n_buf - 1)
            sd = d % n_buf
            pltpu.make_async_copy(table_hbm.at[idx_v.at[sd]], rows_v.at[sd],
                                  sems.at[sd]).wait()
            pltpu.sync_copy(rows_v.at[sd],
                            out_hbm.at[pl.ds(base + d * _CHUNK, _CHUNK)])
    for d in range(n_chunks - (n_buf - 1), n_chunks):
        sd = d % n_buf
        pltpu.make_async_copy(table_hbm.at[idx_v.at[sd]], rows_v.at[sd],
                              sems.at[sd]).wait()
        pltpu.sync_copy(rows_v.at[sd],
                        out_hbm.at[pl.ds(base + d * _CHUNK, _CHUNK)])


def kernel(patch_tokens, voxel_features, voxel_coords, image_sizes, K, Rt,
           W1, b1, W2, b2):
    B, n_cam, M, dim = patch_tokens.shape
    V = voxel_features.shape[1]
    pf_dim = voxel_features.shape[2]
    hidden = W1.shape[1]
    out_dim = W2.shape[1]

    pts_h = jnp.concatenate(
        [voxel_coords, jnp.ones((B, V, 1), voxel_coords.dtype)], axis=-1)
    ptsT = jnp.transpose(pts_h, (0, 2, 1))
    K0 = K[:, 0]
    Rt0 = Rt[:, 0]
    W1a = W1[:pf_dim]
    W1b = W1[pf_dim:]

    smem = pl.BlockSpec(memory_space=pltpu.SMEM)
    table, idx = pl.pallas_call(
        _prep_body,
        grid=(B,),
        in_specs=[
            pl.BlockSpec((1, n_cam, M, dim), lambda b: (b, 0, 0, 0)),
            pl.BlockSpec((1, 4, V), lambda b: (b, 0, 0)),
            pl.BlockSpec((1, 3, 3), lambda b: (b, 0, 0)),
            pl.BlockSpec((1, 4, 4), lambda b: (b, 0, 0)),
            smem,
            pl.BlockSpec((dim, hidden), lambda b: (0, 0)),
            pl.BlockSpec((1, hidden), lambda b: (0, 0)),
        ],
        out_specs=[
            pl.BlockSpec((1, M, hidden), lambda b: (b, 0, 0)),
            pl.BlockSpec((1, 1, V), lambda b: (b, 0, 0)),
        ],
        out_shape=[
            jax.ShapeDtypeStruct((B, M, hidden), jnp.float32),
            jax.ShapeDtypeStruct((B, 1, V), jnp.int32),
        ],
    )(patch_tokens, ptsT, K0, Rt0, image_sizes, W1b,
      b1.reshape(1, hidden))

    n_rows = B * V
    n_workers = 32
    per_w = -(-n_rows // (n_workers * _CHUNK)) * _CHUNK
    n_pad = per_w * n_workers
    n_chunks = per_w // _CHUNK
    n_buf = 2

    idx_flat = idx.reshape(n_rows)
    idx_flat = jnp.concatenate(
        [idx_flat, jnp.zeros((n_pad - n_rows,), jnp.int32)])
    table_flat = table.reshape(B * M, hidden)

    mesh = plsc.VectorSubcoreMesh(core_axis_name="c", subcore_axis_name="s")
    gathered = pl.kernel(
        functools.partial(_sc_gather_body, n_chunks=n_chunks, n_buf=n_buf),
        out_type=jax.ShapeDtypeStruct((n_pad, hidden), jnp.float32),
        mesh=mesh,
        scratch_types=[
            pltpu.VMEM((n_buf, _CHUNK), jnp.int32),
            pltpu.VMEM((n_buf, _CHUNK, hidden), jnp.float32),
            pltpu.SemaphoreType.DMA((n_buf,)),
        ],
    )(table_flat, idx_flat)

    rows_blk = 2000
    n_blk = n_rows // rows_blk
    vox_flat = voxel_features.reshape(n_rows, pf_dim)
    out = pl.pallas_call(
        _mlp_body,
        grid=(n_blk,),
        in_specs=[
            pl.BlockSpec((rows_blk, hidden), lambda i: (i, 0)),
            pl.BlockSpec((rows_blk, pf_dim), lambda i: (i, 0)),
            pl.BlockSpec((pf_dim, hidden), lambda i: (0, 0)),
            pl.BlockSpec((hidden, out_dim), lambda i: (0, 0)),
            pl.BlockSpec((1, out_dim), lambda i: (0, 0)),
        ],
        out_specs=pl.BlockSpec((rows_blk, out_dim), lambda i: (i, 0)),
        out_shape=jax.ShapeDtypeStruct((n_rows, out_dim), jnp.float32),
    )(gathered, vox_flat, W1a, W2, b2.reshape(1, out_dim))

    return out.reshape(B, V, out_dim)


# speedup vs baseline: 4.2989x; 1.0109x over previous
import functools

import jax
import jax.numpy as jnp
from jax import lax
from jax.experimental import pallas as pl
from jax.experimental.pallas import tpu as pltpu
from jax.experimental.pallas import tpu_sc as plsc

_RESIZE = 518.0
_PATCH = 14.0
_GRID = 37


def _prep_body(pt_ref, pts_ref, k_ref, rt_ref, isz_ref, w1b_ref, b1_ref,
               table_ref, idx_ref):
    b = pl.program_id(0)
    tok = pt_ref[0]
    mean_tok = jnp.mean(tok, axis=0)
    table_ref[0] = (
        jnp.dot(mean_tok, w1b_ref[...], preferred_element_type=jnp.float32)
        + b1_ref[...]
    )

    ph = pts_ref[0].astype(jnp.bfloat16)
    rb = rt_ref[0].astype(jnp.bfloat16)
    cam = jnp.dot(rb, ph, preferred_element_type=jnp.float32)
    camb = cam[:3].astype(jnp.bfloat16)
    kb = k_ref[0].astype(jnp.bfloat16)
    pix = jnp.dot(kb, camb, preferred_element_type=jnp.float32)
    denom = pix[2:3, :] + 1e-12
    w_orig = isz_ref[0, 0].astype(jnp.float32)
    h_orig = isz_ref[0, 1].astype(jnp.float32)
    u = (pix[0:1, :] / denom) * (_RESIZE / w_orig)
    v = (pix[1:2, :] / denom) * (_RESIZE / h_orig)
    px = jnp.clip((u / _PATCH).astype(jnp.int32), 0, _GRID - 1)
    py = jnp.clip((v / _PATCH).astype(jnp.int32), 0, _GRID - 1)
    m = pt_ref.shape[2]
    idx_ref[0] = px * _GRID + py + b * m


def _mlp_body(g_ref, vox_ref, w1a_ref, w2_ref, b2_ref, out_ref):
    h = (jnp.dot(vox_ref[...], w1a_ref[...], preferred_element_type=jnp.float32)
         + g_ref[...])
    h = jnp.maximum(h, 0.0)
    out_ref[...] = (
        jnp.dot(h, w2_ref[...], preferred_element_type=jnp.float32)
        + b2_ref[...]
    )


_CHUNK = 128


def _sc_gather_body(table_hbm, idx_hbm, out_hbm, idx_v, rows_v, gsem, osem,
                    n_chunks, n_buf):
    wid = lax.axis_index("s") * 2 + lax.axis_index("c")
    base_c = wid * n_chunks
    pltpu.sync_copy(idx_hbm.at[wid], idx_v)

    def gather(c, s):
        return pltpu.make_async_copy(table_hbm.at[idx_v.at[c]], rows_v.at[s],
                                     gsem.at[s])

    def put(c, s):
        return pltpu.make_async_copy(
            rows_v.at[s], out_hbm.at[pl.ds((base_c + c) * _CHUNK, _CHUNK)],
            osem.at[s])

    for c in range(min(n_buf, n_chunks)):
        gather(c, c).start()
    for c in range(n_chunks):
        s = c % n_buf
        gather(c, s).wait()
        put(c, s).start()
        nc = c + n_buf
        if nc < n_chunks:
            put(c, s).wait()
            gather(nc, s).start()
    for c in range(max(0, n_chunks - n_buf), n_chunks):
        put(c, c % n_buf).wait()


def kernel(patch_tokens, voxel_features, voxel_coords, image_sizes, K, Rt,
           W1, b1, W2, b2):
    B, n_cam, M, dim = patch_tokens.shape
    V = voxel_features.shape[1]
    pf_dim = voxel_features.shape[2]
    hidden = W1.shape[1]
    out_dim = W2.shape[1]

    pts_h = jnp.concatenate(
        [voxel_coords, jnp.ones((B, V, 1), voxel_coords.dtype)], axis=-1)
    ptsT = jnp.transpose(pts_h, (0, 2, 1))
    K0 = K[:, 0]
    Rt0 = Rt[:, 0]
    W1a = W1[:pf_dim]
    W1b = W1[pf_dim:]

    smem = pl.BlockSpec(memory_space=pltpu.SMEM)
    table, idx = pl.pallas_call(
        _prep_body,
        grid=(B,),
        in_specs=[
            pl.BlockSpec((1, n_cam, M, dim), lambda b: (b, 0, 0, 0)),
            pl.BlockSpec((1, 4, V), lambda b: (b, 0, 0)),
            pl.BlockSpec((1, 3, 3), lambda b: (b, 0, 0)),
            pl.BlockSpec((1, 4, 4), lambda b: (b, 0, 0)),
            smem,
            pl.BlockSpec((dim, hidden), lambda b: (0, 0)),
            pl.BlockSpec((1, hidden), lambda b: (0, 0)),
        ],
        out_specs=[
            pl.BlockSpec((1, M, hidden), lambda b: (b, 0, 0)),
            pl.BlockSpec((1, 1, V), lambda b: (b, 0, 0)),
        ],
        out_shape=[
            jax.ShapeDtypeStruct((B, M, hidden), jnp.float32),
            jax.ShapeDtypeStruct((B, 1, V), jnp.int32),
        ],
    )(patch_tokens, ptsT, K0, Rt0, image_sizes, W1b,
      b1.reshape(1, hidden))

    n_rows = B * V
    n_workers = 32
    per_w = -(-n_rows // (n_workers * _CHUNK)) * _CHUNK
    n_pad = per_w * n_workers
    n_chunks = per_w // _CHUNK
    n_buf = 3

    idx_flat = idx.reshape(n_rows)
    idx_flat = jnp.concatenate(
        [idx_flat, jnp.zeros((n_pad - n_rows,), jnp.int32)])
    idx_3d = idx_flat.reshape(n_workers, n_chunks, _CHUNK)
    table_flat = table.reshape(B * M, hidden)

    mesh = plsc.VectorSubcoreMesh(core_axis_name="c", subcore_axis_name="s")
    gathered = pl.kernel(
        functools.partial(_sc_gather_body, n_chunks=n_chunks, n_buf=n_buf),
        out_type=jax.ShapeDtypeStruct((n_pad, hidden), jnp.float32),
        mesh=mesh,
        scratch_types=[
            pltpu.VMEM((n_chunks, _CHUNK), jnp.int32),
            pltpu.VMEM((n_buf, _CHUNK, hidden), jnp.float32),
            pltpu.SemaphoreType.DMA((n_buf,)),
            pltpu.SemaphoreType.DMA((n_buf,)),
        ],
    )(table_flat, idx_3d)

    rows_blk = 2000
    n_blk = n_rows // rows_blk
    vox_flat = voxel_features.reshape(n_rows, pf_dim)
    out = pl.pallas_call(
        _mlp_body,
        grid=(n_blk,),
        in_specs=[
            pl.BlockSpec((rows_blk, hidden), lambda i: (i, 0)),
            pl.BlockSpec((rows_blk, pf_dim), lambda i: (i, 0)),
            pl.BlockSpec((pf_dim, hidden), lambda i: (0, 0)),
            pl.BlockSpec((hidden, out_dim), lambda i: (0, 0)),
            pl.BlockSpec((1, out_dim), lambda i: (0, 0)),
        ],
        out_specs=pl.BlockSpec((rows_blk, out_dim), lambda i: (i, 0)),
        out_shape=jax.ShapeDtypeStruct((n_rows, out_dim), jnp.float32),
    )(gathered, vox_flat, W1a, W2, b2.reshape(1, out_dim))

    return out.reshape(B, V, out_dim)


# speedup vs baseline: 4.4409x; 1.0330x over previous
import functools

import jax
import jax.numpy as jnp
from jax import lax
from jax.experimental import pallas as pl
from jax.experimental.pallas import tpu as pltpu
from jax.experimental.pallas import tpu_sc as plsc

_RESIZE = 518.0
_PATCH = 14.0
_GRID = 37
_CHUNK = 128


def _prep_body(pt_ref, pts_ref, k_ref, rt_ref, isz_ref, w1b_ref, b1_ref,
               table_ref, idx_ref, *, m_pad):
    b = pl.program_id(0)
    tok = pt_ref[0]
    m = tok.shape[1]
    mean_tok = jnp.mean(tok, axis=0)
    res = (jnp.dot(mean_tok, w1b_ref[...], preferred_element_type=jnp.float32)
           + b1_ref[...])
    table_ref[0] = jnp.concatenate(
        [res, jnp.zeros((m_pad - m, res.shape[1]), jnp.float32)], axis=0)

    ph = pts_ref[0].astype(jnp.bfloat16)
    rb = rt_ref[0].astype(jnp.bfloat16)
    cam = jnp.dot(rb, ph, preferred_element_type=jnp.float32)
    camb = cam[:3].astype(jnp.bfloat16)
    kb = k_ref[0].astype(jnp.bfloat16)
    pix = jnp.dot(kb, camb, preferred_element_type=jnp.float32)
    denom = pix[2:3, :] + 1e-12
    w_orig = isz_ref[0, 0].astype(jnp.float32)
    h_orig = isz_ref[0, 1].astype(jnp.float32)
    u = (pix[0:1, :] / denom) * (_RESIZE / w_orig)
    v = (pix[1:2, :] / denom) * (_RESIZE / h_orig)
    px = jnp.clip((u / _PATCH).astype(jnp.int32), 0, _GRID - 1)
    py = jnp.clip((v / _PATCH).astype(jnp.int32), 0, _GRID - 1)
    idx_ref[0] = px * _GRID + py + b * m_pad


def _mlp_body(g_ref, vox_ref, w1a_ref, w2_ref, b2_ref, out_ref):
    h = (jnp.dot(vox_ref[0], w1a_ref[...], preferred_element_type=jnp.float32)
         + g_ref[0])
    h = jnp.maximum(h, 0.0)
    out_ref[0] = (
        jnp.dot(h, w2_ref[...], preferred_element_type=jnp.float32)
        + b2_ref[...]
    )


def _sc_gather_body(table_hbm, idx_hbm, out_hbm, idx_v, rows_v, gsem, osem,
                    *, n_chunks, n_buf, per_w, n_w_per_b):
    wid = lax.axis_index("s") * 2 + lax.axis_index("c")
    b = wid // n_w_per_b
    w16 = wid % n_w_per_b
    off = pl.multiple_of(w16 * per_w, _CHUNK)
    pltpu.sync_copy(idx_hbm.at[b, 0, pl.ds(off, per_w)], idx_v)
    base = wid * per_w

    def gather(c, s):
        return pltpu.make_async_copy(
            table_hbm.at[idx_v.at[pl.ds(c * _CHUNK, _CHUNK)]], rows_v.at[s],
            gsem.at[s])

    def put(c, s):
        return pltpu.make_async_copy(
            rows_v.at[s], out_hbm.at[pl.ds(base + c * _CHUNK, _CHUNK)],
            osem.at[s])

    for c in range(min(n_buf, n_chunks)):
        gather(c, c).start()
    for c in range(n_chunks):
        s = c % n_buf
        gather(c, s).wait()
        put(c, s).start()
        nc = c + n_buf
        if nc < n_chunks:
            put(c, s).wait()
            gather(nc, s).start()
    for c in range(max(0, n_chunks - n_buf), n_chunks):
        put(c, c % n_buf).wait()


def kernel(patch_tokens, voxel_features, voxel_coords, image_sizes, K, Rt,
           W1, b1, W2, b2):
    B, n_cam, M, dim = patch_tokens.shape
    V = voxel_features.shape[1]
    pf_dim = voxel_features.shape[2]
    hidden = W1.shape[1]
    out_dim = W2.shape[1]

    n_workers = 32
    n_w_per_b = n_workers // B
    per_w = -(-V // (n_w_per_b * _CHUNK)) * _CHUNK
    v_pad = per_w * n_w_per_b
    n_chunks = per_w // _CHUNK
    m_pad = -(-M // 8) * 8
    n_buf = 3

    pts_h = jnp.concatenate(
        [voxel_coords, jnp.ones((B, V, 1), voxel_coords.dtype)], axis=-1)
    ptsT = jnp.transpose(pts_h, (0, 2, 1))
    ptsT = jnp.pad(ptsT, ((0, 0), (0, 0), (0, v_pad - V)))
    K0 = K[:, 0]
    Rt0 = Rt[:, 0]
    W1a = W1[:pf_dim]
    W1b = W1[pf_dim:]

    smem = pl.BlockSpec(memory_space=pltpu.SMEM)
    table, idx = pl.pallas_call(
        functools.partial(_prep_body, m_pad=m_pad),
        grid=(B,),
        in_specs=[
            pl.BlockSpec((1, n_cam, M, dim), lambda b: (b, 0, 0, 0)),
            pl.BlockSpec((1, 4, v_pad), lambda b: (b, 0, 0)),
            pl.BlockSpec((1, 3, 3), lambda b: (b, 0, 0)),
            pl.BlockSpec((1, 4, 4), lambda b: (b, 0, 0)),
            smem,
            pl.BlockSpec((dim, hidden), lambda b: (0, 0)),
            pl.BlockSpec((1, hidden), lambda b: (0, 0)),
        ],
        out_specs=[
            pl.BlockSpec((1, m_pad, hidden), lambda b: (b, 0, 0)),
            pl.BlockSpec((1, 1, v_pad), lambda b: (b, 0, 0)),
        ],
        out_shape=[
            jax.ShapeDtypeStruct((B, m_pad, hidden), jnp.float32),
            jax.ShapeDtypeStruct((B, 1, v_pad), jnp.int32),
        ],
    )(patch_tokens, ptsT, K0, Rt0, image_sizes, W1b,
      b1.reshape(1, hidden))

    table_flat = table.reshape(B * m_pad, hidden)

    mesh = plsc.VectorSubcoreMesh(core_axis_name="c", subcore_axis_name="s")
    gathered = pl.kernel(
        functools.partial(_sc_gather_body, n_chunks=n_chunks, n_buf=n_buf,
                          per_w=per_w, n_w_per_b=n_w_per_b),
        out_type=jax.ShapeDtypeStruct((B * v_pad, hidden), jnp.float32),
        mesh=mesh,
        scratch_types=[
            pltpu.VMEM((per_w,), jnp.int32),
            pltpu.VMEM((n_buf, _CHUNK, hidden), jnp.float32),
            pltpu.SemaphoreType.DMA((n_buf,)),
            pltpu.SemaphoreType.DMA((n_buf,)),
        ],
    )(table_flat, idx)

    rows_blk = 2000
    n_blk = V // rows_blk
    g3 = gathered.reshape(B, v_pad, hidden)
    out = pl.pallas_call(
        _mlp_body,
        grid=(B, n_blk),
        in_specs=[
            pl.BlockSpec((1, rows_blk, hidden), lambda b, i: (b, i, 0)),
            pl.BlockSpec((1, rows_blk, pf_dim), lambda b, i: (b, i, 0)),
            pl.BlockSpec((pf_dim, hidden), lambda b, i: (0, 0)),
            pl.BlockSpec((hidden, out_dim), lambda b, i: (0, 0)),
            pl.BlockSpec((1, out_dim), lambda b, i: (0, 0)),
        ],
        out_specs=pl.BlockSpec((1, rows_blk, out_dim), lambda b, i: (b, i, 0)),
        out_shape=jax.ShapeDtypeStruct((B, V, out_dim), jnp.float32),
    )(g3, voxel_features, W1a, W2, b2.reshape(1, out_dim))

    return out


# speedup vs baseline: 4.8941x; 1.1021x over previous
import functools

import jax
import jax.numpy as jnp
from jax import lax
from jax.experimental import pallas as pl
from jax.experimental.pallas import tpu as pltpu
from jax.experimental.pallas import tpu_sc as plsc

_RESIZE = 518.0
_PATCH = 14.0
_GRID = 37
_CHUNK = 128


def _prep_body(pt_ref, pts_ref, k_ref, rt_ref, isz_ref, w1b_ref, b1_ref,
               table_ref, idx_ref, *, m_pad):
    b = pl.program_id(0)
    tok = pt_ref[0]
    m = tok.shape[1]
    mean_tok = jnp.mean(tok, axis=0)
    res = (jnp.dot(mean_tok, w1b_ref[...], preferred_element_type=jnp.float32)
           + b1_ref[...])
    resb = jnp.concatenate(
        [res, jnp.zeros((m_pad - m, res.shape[1]), jnp.float32)],
        axis=0).astype(jnp.bfloat16)
    half = resb.shape[1] // 2
    lo = lax.bitcast_convert_type(resb[:, :half], jnp.uint16).astype(jnp.uint32)
    hi = lax.bitcast_convert_type(resb[:, half:], jnp.uint16).astype(jnp.uint32)
    table_ref[0] = lo | (hi << 16)

    ph = pts_ref[0].astype(jnp.bfloat16)
    rb = rt_ref[0].astype(jnp.bfloat16)
    cam = jnp.dot(rb, ph, preferred_element_type=jnp.float32)
    camb = cam[:3].astype(jnp.bfloat16)
    kb = k_ref[0].astype(jnp.bfloat16)
    pix = jnp.dot(kb, camb, preferred_element_type=jnp.float32)
    denom = pix[2:3, :] + 1e-12
    w_orig = isz_ref[0, 0].astype(jnp.float32)
    h_orig = isz_ref[0, 1].astype(jnp.float32)
    u = (pix[0:1, :] / denom) * (_RESIZE / w_orig)
    v = (pix[1:2, :] / denom) * (_RESIZE / h_orig)
    px = jnp.clip((u / _PATCH).astype(jnp.int32), 0, _GRID - 1)
    py = jnp.clip((v / _PATCH).astype(jnp.int32), 0, _GRID - 1)
    idx_ref[0] = px * _GRID + py + b * m_pad


def _mlp_body(g_ref, vox_ref, w1a_ref, w2_ref, b2_ref, out_ref):
    g = g_ref[0]
    lo = lax.bitcast_convert_type(
        (g & 0xFFFF).astype(jnp.uint16), jnp.bfloat16).astype(jnp.float32)
    hi = lax.bitcast_convert_type(
        (g >> 16).astype(jnp.uint16), jnp.bfloat16).astype(jnp.float32)
    gf = jnp.concatenate([lo, hi], axis=-1)
    h = (jnp.dot(vox_ref[0], w1a_ref[...], preferred_element_type=jnp.float32)
         + gf)
    h = jnp.maximum(h, 0.0)
    out_ref[0] = (
        jnp.dot(h, w2_ref[...], preferred_element_type=jnp.float32)
        + b2_ref[...]
    )


def _sc_gather_body(table_hbm, idx_hbm, out_hbm, idx_v, rows_v, gsem, osem,
                    *, n_chunks, n_buf, per_w, n_w_per_b):
    wid = lax.axis_index("s") * 2 + lax.axis_index("c")
    b = wid // n_w_per_b
    w16 = wid % n_w_per_b
    off = pl.multiple_of(w16 * per_w, _CHUNK)
    pltpu.sync_copy(idx_hbm.at[b, 0, pl.ds(off, per_w)], idx_v)
    base = wid * per_w

    def gather(c, s):
        return pltpu.make_async_copy(
            table_hbm.at[idx_v.at[pl.ds(c * _CHUNK, _CHUNK)]], rows_v.at[s],
            gsem.at[s])

    def put(c, s):
        return pltpu.make_async_copy(
            rows_v.at[s], out_hbm.at[pl.ds(base + c * _CHUNK, _CHUNK)],
            osem.at[s])

    for c in range(min(n_buf, n_chunks)):
        gather(c, c).start()
    for c in range(n_chunks):
        s = c % n_buf
        gather(c, s).wait()
        put(c, s).start()
        nc = c + n_buf
        if nc < n_chunks:
            put(c, s).wait()
            gather(nc, s).start()
    for c in range(max(0, n_chunks - n_buf), n_chunks):
        put(c, c % n_buf).wait()


def kernel(patch_tokens, voxel_features, voxel_coords, image_sizes, K, Rt,
           W1, b1, W2, b2):
    B, n_cam, M, dim = patch_tokens.shape
    V = voxel_features.shape[1]
    pf_dim = voxel_features.shape[2]
    hidden = W1.shape[1]
    out_dim = W2.shape[1]

    n_workers = 32
    n_w_per_b = n_workers // B
    per_w = -(-V // (n_w_per_b * _CHUNK)) * _CHUNK
    v_pad = per_w * n_w_per_b
    n_chunks = per_w // _CHUNK
    m_pad = -(-M // 8) * 8
    n_buf = 3

    pts_h = jnp.concatenate(
        [voxel_coords, jnp.ones((B, V, 1), voxel_coords.dtype)], axis=-1)
    ptsT = jnp.transpose(pts_h, (0, 2, 1))
    ptsT = jnp.pad(ptsT, ((0, 0), (0, 0), (0, v_pad - V)))
    K0 = K[:, 0]
    Rt0 = Rt[:, 0]
    W1a = W1[:pf_dim]
    W1b = W1[pf_dim:]

    smem = pl.BlockSpec(memory_space=pltpu.SMEM)
    table, idx = pl.pallas_call(
        functools.partial(_prep_body, m_pad=m_pad),
        grid=(B,),
        in_specs=[
            pl.BlockSpec((1, n_cam, M, dim), lambda b: (b, 0, 0, 0)),
            pl.BlockSpec((1, 4, v_pad), lambda b: (b, 0, 0)),
            pl.BlockSpec((1, 3, 3), lambda b: (b, 0, 0)),
            pl.BlockSpec((1, 4, 4), lambda b: (b, 0, 0)),
            smem,
            pl.BlockSpec((dim, hidden), lambda b: (0, 0)),
            pl.BlockSpec((1, hidden), lambda b: (0, 0)),
        ],
        out_specs=[
            pl.BlockSpec((1, m_pad, hidden // 2), lambda b: (b, 0, 0)),
            pl.BlockSpec((1, 1, v_pad), lambda b: (b, 0, 0)),
        ],
        out_shape=[
            jax.ShapeDtypeStruct((B, m_pad, hidden // 2), jnp.uint32),
            jax.ShapeDtypeStruct((B, 1, v_pad), jnp.int32),
        ],
    )(patch_tokens, ptsT, K0, Rt0, image_sizes, W1b,
      b1.reshape(1, hidden))

    table_flat = table.reshape(B * m_pad, hidden // 2)

    mesh = plsc.VectorSubcoreMesh(core_axis_name="c", subcore_axis_name="s")
    gathered = pl.kernel(
        functools.partial(_sc_gather_body, n_chunks=n_chunks, n_buf=n_buf,
                          per_w=per_w, n_w_per_b=n_w_per_b),
        out_type=jax.ShapeDtypeStruct((B * v_pad, hidden // 2), jnp.uint32),
        mesh=mesh,
        scratch_types=[
            pltpu.VMEM((per_w,), jnp.int32),
            pltpu.VMEM((n_buf, _CHUNK, hidden // 2), jnp.uint32),
            pltpu.SemaphoreType.DMA((n_buf,)),
            pltpu.SemaphoreType.DMA((n_buf,)),
        ],
    )(table_flat, idx)

    rows_blk = 2000
    n_blk = V // rows_blk
    g3 = gathered.reshape(B, v_pad, hidden // 2)
    out = pl.pallas_call(
        _mlp_body,
        grid=(B, n_blk),
        in_specs=[
            pl.BlockSpec((1, rows_blk, hidden // 2), lambda b, i: (b, i, 0)),
            pl.BlockSpec((1, rows_blk, pf_dim), lambda b, i: (b, i, 0)),
            pl.BlockSpec((pf_dim, hidden), lambda b, i: (0, 0)),
            pl.BlockSpec((hidden, out_dim), lambda b, i: (0, 0)),
            pl.BlockSpec((1, out_dim), lambda b, i: (0, 0)),
        ],
        out_specs=pl.BlockSpec((1, rows_blk, out_dim), lambda b, i: (b, i, 0)),
        out_shape=jax.ShapeDtypeStruct((B, V, out_dim), jnp.float32),
    )(g3, voxel_features, W1a, W2, b2.reshape(1, out_dim))

    return out


# speedup vs baseline: 4.9367x; 1.0087x over previous
import functools

import jax
import jax.numpy as jnp
from jax import lax
from jax.experimental import pallas as pl
from jax.experimental.pallas import tpu as pltpu
from jax.experimental.pallas import tpu_sc as plsc

_RESIZE = 518.0
_PATCH = 14.0
_GRID = 37
_CHUNK = 128


def _prep_body(pt_ref, pts_ref, k_ref, rt_ref, isz_ref, w1b_ref, b1_ref,
               table_ref, idx_ref, *, m_pad):
    b = pl.program_id(0)
    tok = pt_ref[0]
    m = tok.shape[1]
    mean_tok = jnp.mean(tok, axis=0)
    res = (jnp.dot(mean_tok, w1b_ref[...], preferred_element_type=jnp.float32)
           + b1_ref[...])
    resb = jnp.concatenate(
        [res, jnp.zeros((m_pad - m, res.shape[1]), jnp.float32)],
        axis=0).astype(jnp.bfloat16)
    half = resb.shape[1] // 2
    lo = lax.bitcast_convert_type(resb[:, :half], jnp.uint16).astype(jnp.uint32)
    hi = lax.bitcast_convert_type(resb[:, half:], jnp.uint16).astype(jnp.uint32)
    table_ref[0] = lo | (hi << 16)

    ph = pts_ref[0].astype(jnp.bfloat16)
    rb = rt_ref[0].astype(jnp.bfloat16)
    cam = jnp.dot(rb, ph, preferred_element_type=jnp.float32)
    camb = cam[:3].astype(jnp.bfloat16)
    kb = k_ref[0].astype(jnp.bfloat16)
    pix = jnp.dot(kb, camb, preferred_element_type=jnp.float32)
    denom = pix[2:3, :] + 1e-12
    w_orig = isz_ref[0, 0].astype(jnp.float32)
    h_orig = isz_ref[0, 1].astype(jnp.float32)
    u = (pix[0:1, :] / denom) * (_RESIZE / w_orig)
    v = (pix[1:2, :] / denom) * (_RESIZE / h_orig)
    px = jnp.clip((u / _PATCH).astype(jnp.int32), 0, _GRID - 1)
    py = jnp.clip((v / _PATCH).astype(jnp.int32), 0, _GRID - 1)
    idx_ref[0] = px * _GRID + py + b * m_pad


def _mlp_body(g_ref, vox_ref, w1a_ref, w2_ref, b2_ref, out_ref):
    g = g_ref[0]
    lo = lax.bitcast_convert_type(
        (g & 0xFFFF).astype(jnp.uint16), jnp.bfloat16).astype(jnp.float32)
    hi = lax.bitcast_convert_type(
        (g >> 16).astype(jnp.uint16), jnp.bfloat16).astype(jnp.float32)
    gf = jnp.concatenate([lo, hi], axis=-1)
    h = (jnp.dot(vox_ref[0], w1a_ref[...], preferred_element_type=jnp.float32)
         + gf)
    h = jnp.maximum(h, 0.0)
    out_ref[0] = (
        jnp.dot(h, w2_ref[...], preferred_element_type=jnp.float32)
        + b2_ref[...]
    )


def _sc_gather_body(table_hbm, idx_hbm, out_hbm, idx_v, rows_v, gsem, osem,
                    *, n_chunks, n_buf, per_w, n_w_per_b):
    wid = lax.axis_index("s") * 2 + lax.axis_index("c")
    b = wid // n_w_per_b
    w16 = wid % n_w_per_b
    off = pl.multiple_of(w16 * per_w, _CHUNK)
    pltpu.sync_copy(idx_hbm.at[b, 0, pl.ds(off, per_w)], idx_v)
    base = wid * per_w

    def gather(c, s):
        return pltpu.make_async_copy(
            table_hbm.at[idx_v.at[pl.ds(c * _CHUNK, _CHUNK)]], rows_v.at[s],
            gsem.at[s])

    def put(c, s):
        return pltpu.make_async_copy(
            rows_v.at[s], out_hbm.at[pl.ds(base + c * _CHUNK, _CHUNK)],
            osem.at[s])

    for c in range(min(n_buf, n_chunks)):
        gather(c, c).start()
    for c in range(n_chunks):
        s = c % n_buf
        gather(c, s).wait()
        put(c, s).start()
        nc = c + n_buf
        if nc < n_chunks:
            put(c, s).wait()
            gather(nc, s).start()
    for c in range(max(0, n_chunks - n_buf), n_chunks):
        put(c, c % n_buf).wait()


def kernel(patch_tokens, voxel_features, voxel_coords, image_sizes, K, Rt,
           W1, b1, W2, b2):
    B, n_cam, M, dim = patch_tokens.shape
    V = voxel_features.shape[1]
    pf_dim = voxel_features.shape[2]
    hidden = W1.shape[1]
    out_dim = W2.shape[1]

    n_workers = 32
    n_w_per_b = n_workers // B
    per_w = -(-V // (n_w_per_b * _CHUNK)) * _CHUNK
    v_pad = per_w * n_w_per_b
    n_chunks = per_w // _CHUNK
    m_pad = -(-M // 8) * 8
    n_buf = 6

    pts_h = jnp.concatenate(
        [voxel_coords, jnp.ones((B, V, 1), voxel_coords.dtype)], axis=-1)
    ptsT = jnp.transpose(pts_h, (0, 2, 1))
    ptsT = jnp.pad(ptsT, ((0, 0), (0, 0), (0, v_pad - V)))
    K0 = K[:, 0]
    Rt0 = Rt[:, 0]
    W1a = W1[:pf_dim]
    W1b = W1[pf_dim:]

    smem = pl.BlockSpec(memory_space=pltpu.SMEM)
    table, idx = pl.pallas_call(
        functools.partial(_prep_body, m_pad=m_pad),
        grid=(B,),
        in_specs=[
            pl.BlockSpec((1, n_cam, M, dim), lambda b: (b, 0, 0, 0)),
            pl.BlockSpec((1, 4, v_pad), lambda b: (b, 0, 0)),
            pl.BlockSpec((1, 3, 3), lambda b: (b, 0, 0)),
            pl.BlockSpec((1, 4, 4), lambda b: (b, 0, 0)),
            smem,
            pl.BlockSpec((dim, hidden), lambda b: (0, 0)),
            pl.BlockSpec((1, hidden), lambda b: (0, 0)),
        ],
        out_specs=[
            pl.BlockSpec((1, m_pad, hidden // 2), lambda b: (b, 0, 0)),
            pl.BlockSpec((1, 1, v_pad), lambda b: (b, 0, 0)),
        ],
        out_shape=[
            jax.ShapeDtypeStruct((B, m_pad, hidden // 2), jnp.uint32),
            jax.ShapeDtypeStruct((B, 1, v_pad), jnp.int32),
        ],
    )(patch_tokens, ptsT, K0, Rt0, image_sizes, W1b,
      b1.reshape(1, hidden))

    table_flat = table.reshape(B * m_pad, hidden // 2)

    mesh = plsc.VectorSubcoreMesh(core_axis_name="c", subcore_axis_name="s")
    gathered = pl.kernel(
        functools.partial(_sc_gather_body, n_chunks=n_chunks, n_buf=n_buf,
                          per_w=per_w, n_w_per_b=n_w_per_b),
        out_type=jax.ShapeDtypeStruct((B * v_pad, hidden // 2), jnp.uint32),
        mesh=mesh,
        scratch_types=[
            pltpu.VMEM((per_w,), jnp.int32),
            pltpu.VMEM((n_buf, _CHUNK, hidden // 2), jnp.uint32),
            pltpu.SemaphoreType.DMA((n_buf,)),
            pltpu.SemaphoreType.DMA((n_buf,)),
        ],
    )(table_flat, idx)

    rows_blk = 2000
    n_blk = V // rows_blk
    g3 = gathered.reshape(B, v_pad, hidden // 2)
    out = pl.pallas_call(
        _mlp_body,
        grid=(B, n_blk),
        in_specs=[
            pl.BlockSpec((1, rows_blk, hidden // 2), lambda b, i: (b, i, 0)),
            pl.BlockSpec((1, rows_blk, pf_dim), lambda b, i: (b, i, 0)),
            pl.BlockSpec((pf_dim, hidden), lambda b, i: (0, 0)),
            pl.BlockSpec((hidden, out_dim), lambda b, i: (0, 0)),
            pl.BlockSpec((1, out_dim), lambda b, i: (0, 0)),
        ],
        out_specs=pl.BlockSpec((1, rows_blk, out_dim), lambda b, i: (b, i, 0)),
        out_shape=jax.ShapeDtypeStruct((B, V, out_dim), jnp.float32),
    )(g3, voxel_features, W1a, W2, b2.reshape(1, out_dim))

    return out


# speedup vs baseline: 11.6479x; 2.3594x over previous
import functools

import jax
import jax.numpy as jnp
from jax import lax
from jax.experimental import pallas as pl
from jax.experimental.pallas import tpu as pltpu
from jax.experimental.pallas import tpu_sc as plsc

_RESIZE = 518.0
_PATCH = 14.0
_GRID = 37
_CHUNK = 128


def _prep_body(pt_ref, pts_ref, k_ref, rt_ref, isz_ref, w1b_ref, b1_ref,
               table_ref, idx_ref, *, m_pad):
    b = pl.program_id(0)
    tok = pt_ref[0]
    m = tok.shape[1]
    mean_tok = jnp.mean(tok, axis=0)
    res = (jnp.dot(mean_tok, w1b_ref[...], preferred_element_type=jnp.float32)
           + b1_ref[...])
    resb = jnp.concatenate(
        [res, jnp.zeros((m_pad - m, res.shape[1]), jnp.float32)],
        axis=0).astype(jnp.bfloat16)
    half = resb.shape[1] // 2
    lo = lax.bitcast_convert_type(resb[:, :half], jnp.uint16).astype(jnp.uint32)
    hi = lax.bitcast_convert_type(resb[:, half:], jnp.uint16).astype(jnp.uint32)
    table_ref[0] = lo | (hi << 16)

    ph = pts_ref[0].astype(jnp.bfloat16)
    rb = rt_ref[0].astype(jnp.bfloat16)
    cam = jnp.dot(rb, ph, preferred_element_type=jnp.float32)
    camb = cam[:3].astype(jnp.bfloat16)
    kb = k_ref[0].astype(jnp.bfloat16)
    pix = jnp.dot(kb, camb, preferred_element_type=jnp.float32)
    denom = pix[2:3, :] + 1e-12
    w_orig = isz_ref[0, 0].astype(jnp.float32)
    h_orig = isz_ref[0, 1].astype(jnp.float32)
    u = (pix[0:1, :] / denom) * (_RESIZE / w_orig)
    v = (pix[1:2, :] / denom) * (_RESIZE / h_orig)
    px = jnp.clip((u / _PATCH).astype(jnp.int32), 0, _GRID - 1)
    py = jnp.clip((v / _PATCH).astype(jnp.int32), 0, _GRID - 1)
    idx_ref[0] = px * _GRID + py + b * m_pad


def _mlp_body(g_ref, vox_ref, w1a_ref, w2_ref, b2_ref, out_ref):
    g = g_ref[0]
    lo = lax.bitcast_convert_type(
        (g & 0xFFFF).astype(jnp.uint16), jnp.bfloat16).astype(jnp.float32)
    hi = lax.bitcast_convert_type(
        (g >> 16).astype(jnp.uint16), jnp.bfloat16).astype(jnp.float32)
    gf = jnp.concatenate([lo, hi], axis=-1)
    h = (jnp.dot(vox_ref[0], w1a_ref[...], preferred_element_type=jnp.float32)
         + gf)
    h = jnp.maximum(h, 0.0)
    out_ref[0] = (
        jnp.dot(h, w2_ref[...], preferred_element_type=jnp.float32)
        + b2_ref[...]
    )


def _sc_gather_body(table_hbm, idx_hbm, out_hbm, idx_v, rows_v, table_sh,
                    gsem, osem, *, n_chunks, n_buf, per_w, n_w_per_b):
    sid = lax.axis_index("s")
    wid = sid * 2 + lax.axis_index("c")
    b = wid // n_w_per_b
    w16 = wid % n_w_per_b

    @pl.when(sid == 0)
    def _():
        pltpu.sync_copy(table_hbm, table_sh)

    off = pl.multiple_of(w16 * per_w, _CHUNK)
    pltpu.sync_copy(idx_hbm.at[b, 0, pl.ds(off, per_w)], idx_v)
    plsc.subcore_barrier()
    base = wid * per_w

    def gather(c, s):
        return pltpu.make_async_copy(
            table_sh.at[idx_v.at[pl.ds(c * _CHUNK, _CHUNK)]], rows_v.at[s],
            gsem.at[s])

    def put(c, s):
        return pltpu.make_async_copy(
            rows_v.at[s], out_hbm.at[pl.ds(base + c * _CHUNK, _CHUNK)],
            osem.at[s])

    for c in range(min(n_buf, n_chunks)):
        gather(c, c).start()
    for c in range(n_chunks):
        s = c % n_buf
        gather(c, s).wait()
        put(c, s).start()
        nc = c + n_buf
        if nc < n_chunks:
            put(c, s).wait()
            gather(nc, s).start()
    for c in range(max(0, n_chunks - n_buf), n_chunks):
        put(c, c % n_buf).wait()


def kernel(patch_tokens, voxel_features, voxel_coords, image_sizes, K, Rt,
           W1, b1, W2, b2):
    B, n_cam, M, dim = patch_tokens.shape
    V = voxel_features.shape[1]
    pf_dim = voxel_features.shape[2]
    hidden = W1.shape[1]
    out_dim = W2.shape[1]

    n_workers = 32
    n_w_per_b = n_workers // B
    per_w = -(-V // (n_w_per_b * _CHUNK)) * _CHUNK
    v_pad = per_w * n_w_per_b
    n_chunks = per_w // _CHUNK
    m_pad = -(-M // 8) * 8
    n_buf = 6

    pts_h = jnp.concatenate(
        [voxel_coords, jnp.ones((B, V, 1), voxel_coords.dtype)], axis=-1)
    ptsT = jnp.transpose(pts_h, (0, 2, 1))
    ptsT = jnp.pad(ptsT, ((0, 0), (0, 0), (0, v_pad - V)))
    K0 = K[:, 0]
    Rt0 = Rt[:, 0]
    W1a = W1[:pf_dim]
    W1b = W1[pf_dim:]

    smem = pl.BlockSpec(memory_space=pltpu.SMEM)
    table, idx = pl.pallas_call(
        functools.partial(_prep_body, m_pad=m_pad),
        grid=(B,),
        in_specs=[
            pl.BlockSpec((1, n_cam, M, dim), lambda b: (b, 0, 0, 0)),
            pl.BlockSpec((1, 4, v_pad), lambda b: (b, 0, 0)),
            pl.BlockSpec((1, 3, 3), lambda b: (b, 0, 0)),
            pl.BlockSpec((1, 4, 4), lambda b: (b, 0, 0)),
            smem,
            pl.BlockSpec((dim, hidden), lambda b: (0, 0)),
            pl.BlockSpec((1, hidden), lambda b: (0, 0)),
        ],
        out_specs=[
            pl.BlockSpec((1, m_pad, hidden // 2), lambda b: (b, 0, 0)),
            pl.BlockSpec((1, 1, v_pad), lambda b: (b, 0, 0)),
        ],
        out_shape=[
            jax.ShapeDtypeStruct((B, m_pad, hidden // 2), jnp.uint32),
            jax.ShapeDtypeStruct((B, 1, v_pad), jnp.int32),
        ],
    )(patch_tokens, ptsT, K0, Rt0, image_sizes, W1b,
      b1.reshape(1, hidden))

    table_flat = table.reshape(B * m_pad, hidden // 2)

    mesh = plsc.VectorSubcoreMesh(core_axis_name="c", subcore_axis_name="s")
    gathered = pl.kernel(
        functools.partial(_sc_gather_body, n_chunks=n_chunks, n_buf=n_buf,
                          per_w=per_w, n_w_per_b=n_w_per_b),
        out_type=jax.ShapeDtypeStruct((B * v_pad, hidden // 2), jnp.uint32),
        mesh=mesh,
        scratch_types=[
            pltpu.VMEM((per_w,), jnp.int32),
            pltpu.VMEM((n_buf, _CHUNK, hidden // 2), jnp.uint32),
            pltpu.VMEM_SHARED((B * m_pad, hidden // 2), jnp.uint32),
            pltpu.SemaphoreType.DMA((n_buf,)),
            pltpu.SemaphoreType.DMA((n_buf,)),
        ],
    )(table_flat, idx)

    rows_blk = 2000
    n_blk = V // rows_blk
    g3 = gathered.reshape(B, v_pad, hidden // 2)
    out = pl.pallas_call(
        _mlp_body,
        grid=(B, n_blk),
        in_specs=[
            pl.BlockSpec((1, rows_blk, hidden // 2), lambda b, i: (b, i, 0)),
            pl.BlockSpec((1, rows_blk, pf_dim), lambda b, i: (b, i, 0)),
            pl.BlockSpec((pf_dim, hidden), lambda b, i: (0, 0)),
            pl.BlockSpec((hidden, out_dim), lambda b, i: (0, 0)),
            pl.BlockSpec((1, out_dim), lambda b, i: (0, 0)),
        ],
        out_specs=pl.BlockSpec((1, rows_blk, out_dim), lambda b, i: (b, i, 0)),
        out_shape=jax.ShapeDtypeStruct((B, V, out_dim), jnp.float32),
    )(g3, voxel_features, W1a, W2, b2.reshape(1, out_dim))

    return out
